# baseline (speedup 1.0000x reference)
import jax
import jax.numpy as jnp
from jax.experimental import pallas as pl

_NC = 1000


def _onehot_block(x_ref, o_ref):
    xv = x_ref[...]
    iota = jax.lax.broadcasted_iota(jnp.int32, o_ref.shape, 2)
    o_ref[...] = (xv[:, :, None] == iota).astype(jnp.float32)


def kernel(x):
    B, S = x.shape
    R = 128
    return pl.pallas_call(
        _onehot_block,
        grid=(B // R,),
        in_specs=[pl.BlockSpec((R, S), lambda i: (i, 0))],
        out_specs=pl.BlockSpec((R, S, _NC), lambda i: (i, 0, 0)),
        out_shape=jax.ShapeDtypeStruct((B, S, _NC), jnp.float32),
    )(x)
